# Initial kernel scaffold; baseline (speedup 1.0000x reference)
import jax
import jax.numpy as jnp
from jax.experimental import pallas as pl


def kernel(features, edge_index, w1_self, w1_neigh, w2_self, w2_neigh, w_fc1):
    raise NotImplementedError("write your pallas kernel here")


# speedup vs baseline: 7.2975x; 7.2975x over previous
import functools

import jax
import jax.numpy as jnp
from jax import lax
from jax.experimental import pallas as pl
from jax.experimental.pallas import tpu as pltpu
from jax.experimental.pallas import tpu_sc as plsc

_N = 10000
_E = 320000
_D = 128
_H = 10
_L = 16
_NC = 2
_NS = 16
_NW = _NC * _NS
_NPAD = 10240
_RPT = _NPAD // _NS
_EPW = _E // _NW
_CHUNK = 80
_NCHUNKS = _EPW // _CHUNK


def _project_body(feat_ref, wn_ref, ws_ref, pn_ref, ps_ref):
    f = feat_ref[...]
    pn = jnp.dot(f, wn_ref[...], preferred_element_type=jnp.float32)
    ps = jnp.dot(f, ws_ref[...], preferred_element_type=jnp.float32)
    lane = lax.broadcasted_iota(jnp.int32, pn.shape, 1)
    pn_ref[...] = jnp.where(lane == _H, 1.0, pn)
    ps_ref[...] = ps


def _project(feat_pad, wn, ws):
    bm = 1024
    return pl.pallas_call(
        _project_body,
        grid=(_NPAD // bm,),
        in_specs=[
            pl.BlockSpec((bm, _D), lambda i: (i, 0)),
            pl.BlockSpec((_D, _L), lambda i: (0, 0)),
            pl.BlockSpec((_D, _L), lambda i: (0, 0)),
        ],
        out_specs=[
            pl.BlockSpec((bm, _L), lambda i: (i, 0)),
            pl.BlockSpec((bm, _L), lambda i: (i, 0)),
        ],
        out_shape=[
            jax.ShapeDtypeStruct((_NPAD, _L), jnp.float32),
            jax.ShapeDtypeStruct((_NPAD, _L), jnp.float32),
        ],
    )(feat_pad, wn, ws)


def _x1_body(ps_ref, acc_ref, x1_ref):
    a = acc_ref[0] + acc_ref[1]
    lane = lax.broadcasted_iota(jnp.int32, a.shape, 1)
    deg = jnp.sum(jnp.where(lane == _H, a, 0.0), axis=1, keepdims=True)
    invd = 1.0 / jnp.maximum(deg, 1.0)
    x1 = jnp.maximum(ps_ref[...] + a * invd, 0.0)
    x1_ref[...] = jnp.where(lane < _H, x1, 0.0)


def _x1(ps, acc1):
    bm = 1024
    return pl.pallas_call(
        _x1_body,
        grid=(_NPAD // bm,),
        in_specs=[
            pl.BlockSpec((bm, _L), lambda i: (i, 0)),
            pl.BlockSpec((_NC, bm, _L), lambda i: (0, i, 0)),
        ],
        out_specs=pl.BlockSpec((bm, _L), lambda i: (i, 0)),
        out_shape=jax.ShapeDtypeStruct((_NPAD, _L), jnp.float32),
    )(ps, acc1)


def _final_body(x1_ref, acc1_ref, acc2_ref, w2s_ref, w2n_ref, wfc_ref, out_ref):
    a1 = acc1_ref[0] + acc1_ref[1]
    a2 = acc2_ref[0] + acc2_ref[1]
    lane = lax.broadcasted_iota(jnp.int32, a1.shape, 1)
    deg = jnp.sum(jnp.where(lane == _H, a1, 0.0), axis=1, keepdims=True)
    invd = 1.0 / jnp.maximum(deg, 1.0)
    a_vec = jnp.sum(x1_ref[...], axis=0, keepdims=True)
    b_vec = jnp.sum(a2 * invd, axis=0, keepdims=True)
    g = (jnp.dot(a_vec, w2s_ref[...], preferred_element_type=jnp.float32)
         + jnp.dot(b_vec, w2n_ref[...], preferred_element_type=jnp.float32))
    g = g * (1.0 / _N)
    o = jnp.dot(g, wfc_ref[...], preferred_element_type=jnp.float32)
    out_ref[...] = jax.nn.sigmoid(o[:, :1])


def _final(x1, acc1, acc2, w2s, w2n, wfc):
    return pl.pallas_call(
        _final_body,
        out_shape=jax.ShapeDtypeStruct((1, 1), jnp.float32),
    )(x1, acc1, acc2, w2s, w2n, wfc)


def _edge_body(table_hbm, src_hbm, dst_hbm, zeros_hbm, out_hbm,
               src_v, dst_v, rows_v, acc_sh, sem):
    cid = lax.axis_index("c")
    sid = lax.axis_index("s")
    rbase = sid * _RPT
    pltpu.sync_copy(zeros_hbm.at[pl.ds(rbase, _RPT)],
                    acc_sh.at[pl.ds(rbase, _RPT)])
    plsc.subcore_barrier()

    ebase = (cid * _NS + sid) * _EPW

    def chunk(i, carry):
        off = ebase + i * _CHUNK
        pltpu.sync_copy(src_hbm.at[pl.ds(off, _CHUNK)], src_v)
        pltpu.sync_copy(dst_hbm.at[pl.ds(off, _CHUNK)], dst_v)
        pltpu.async_copy(table_hbm.at[src_v], rows_v, sem).wait()
        pltpu.sync_copy(rows_v, acc_sh.at[dst_v], add=True)
        return carry

    lax.fori_loop(0, _NCHUNKS, chunk, 0)
    plsc.subcore_barrier()
    pltpu.sync_copy(acc_sh.at[pl.ds(rbase, _RPT)],
                    out_hbm.at[pl.ds(cid * _NPAD + rbase, _RPT)])


@functools.cache
def _edge_pass_call():
    return pl.kernel(
        _edge_body,
        out_type=jax.ShapeDtypeStruct((_NC * _NPAD, _L), jnp.float32),
        mesh=plsc.VectorSubcoreMesh(core_axis_name="c", subcore_axis_name="s",
                                    num_cores=_NC, num_subcores=_NS),
        scratch_types=[
            pltpu.VMEM((_CHUNK,), jnp.int32),
            pltpu.VMEM((_CHUNK,), jnp.int32),
            pltpu.VMEM((_CHUNK, _L), jnp.float32),
            pltpu.VMEM_SHARED((_NPAD, _L), jnp.float32),
            pltpu.SemaphoreType.DMA,
        ],
        compiler_params=pltpu.CompilerParams(use_tc_tiling_on_sc=False),
    )


def _edge_pass(table, src, dst, ztbl):
    return _edge_pass_call()(table, src, dst, ztbl)


def kernel(features, edge_index, w1_self, w1_neigh, w2_self, w2_neigh, w_fc1):
    src = edge_index[0]
    dst = edge_index[1]
    feat_pad = jnp.pad(features, ((0, _NPAD - _N), (0, 0)))
    wn = jnp.pad(w1_neigh, ((0, 0), (0, _L - _H)))
    ws = jnp.pad(w1_self, ((0, 0), (0, _L - _H)))
    w2s = jnp.pad(w2_self, ((0, _L - _H), (0, _L - _H)))
    w2n = jnp.pad(w2_neigh, ((0, _L - _H), (0, _L - _H)))
    wfc = jnp.pad(w_fc1, ((0, _L - _H), (0, 127)))
    ztbl = jnp.zeros((_NPAD, _L), jnp.float32)

    pn, ps = _project(feat_pad, wn, ws)
    acc1 = _edge_pass(pn, src, dst, ztbl).reshape(_NC, _NPAD, _L)
    x1 = _x1(ps, acc1)
    acc2 = _edge_pass(x1, src, dst, ztbl).reshape(_NC, _NPAD, _L)
    return _final(x1, acc1, acc2, w2s, w2n, wfc)


# speedup vs baseline: 13.4245x; 1.8396x over previous
import functools

import jax
import jax.numpy as jnp
from jax import lax
from jax.experimental import pallas as pl
from jax.experimental.pallas import tpu as pltpu
from jax.experimental.pallas import tpu_sc as plsc

_N = 10000
_E = 320000
_D = 128
_H = 10
_L = 16
_NC = 2
_NS = 16
_NW = _NC * _NS
_NPAD = 10240
_RPT = _NPAD // _NS
_EPW = _E // _NW
_CHUNK = 80
_NCHUNKS = _EPW // _CHUNK


def _project_body(feat_ref, wn_ref, ws_ref, pn_ref, ps_ref):
    f = feat_ref[...]
    pn = jnp.dot(f, wn_ref[...], preferred_element_type=jnp.float32)
    ps = jnp.dot(f, ws_ref[...], preferred_element_type=jnp.float32)
    lane = lax.broadcasted_iota(jnp.int32, pn.shape, 1)
    pn_ref[...] = jnp.where(lane == _H, 1.0, pn)
    ps_ref[...] = ps


def _project(feat_pad, wn, ws):
    bm = 1024
    return pl.pallas_call(
        _project_body,
        grid=(_NPAD // bm,),
        in_specs=[
            pl.BlockSpec((bm, _D), lambda i: (i, 0)),
            pl.BlockSpec((_D, _L), lambda i: (0, 0)),
            pl.BlockSpec((_D, _L), lambda i: (0, 0)),
        ],
        out_specs=[
            pl.BlockSpec((bm, _L), lambda i: (i, 0)),
            pl.BlockSpec((bm, _L), lambda i: (i, 0)),
        ],
        out_shape=[
            jax.ShapeDtypeStruct((_NPAD, _L), jnp.float32),
            jax.ShapeDtypeStruct((_NPAD, _L), jnp.float32),
        ],
    )(feat_pad, wn, ws)


def _x1_body(ps_ref, acc_ref, x1_ref):
    a = acc_ref[0] + acc_ref[1]
    lane = lax.broadcasted_iota(jnp.int32, a.shape, 1)
    deg = jnp.sum(jnp.where(lane == _H, a, 0.0), axis=1, keepdims=True)
    invd = 1.0 / jnp.maximum(deg, 1.0)
    x1 = jnp.maximum(ps_ref[...] + a * invd, 0.0)
    x1_ref[...] = jnp.where(lane < _H, x1, 0.0)


def _x1(ps, acc1):
    bm = 1024
    return pl.pallas_call(
        _x1_body,
        grid=(_NPAD // bm,),
        in_specs=[
            pl.BlockSpec((bm, _L), lambda i: (i, 0)),
            pl.BlockSpec((_NC, bm, _L), lambda i: (0, i, 0)),
        ],
        out_specs=pl.BlockSpec((bm, _L), lambda i: (i, 0)),
        out_shape=jax.ShapeDtypeStruct((_NPAD, _L), jnp.float32),
    )(ps, acc1)


def _final_body(x1_ref, acc1_ref, acc2_ref, w2s_ref, w2n_ref, wfc_ref, out_ref):
    a1 = acc1_ref[0] + acc1_ref[1]
    a2 = acc2_ref[0] + acc2_ref[1]
    lane = lax.broadcasted_iota(jnp.int32, a1.shape, 1)
    deg = jnp.sum(jnp.where(lane == _H, a1, 0.0), axis=1, keepdims=True)
    invd = 1.0 / jnp.maximum(deg, 1.0)
    a_vec = jnp.sum(x1_ref[...], axis=0, keepdims=True)
    b_vec = jnp.sum(a2 * invd, axis=0, keepdims=True)
    g = (jnp.dot(a_vec, w2s_ref[...], preferred_element_type=jnp.float32)
         + jnp.dot(b_vec, w2n_ref[...], preferred_element_type=jnp.float32))
    g = g * (1.0 / _N)
    o = jnp.dot(g, wfc_ref[...], preferred_element_type=jnp.float32)
    out_ref[...] = jax.nn.sigmoid(o[:, :1])


def _final(x1, acc1, acc2, w2s, w2n, wfc):
    return pl.pallas_call(
        _final_body,
        out_shape=jax.ShapeDtypeStruct((1, 1), jnp.float32),
    )(x1, acc1, acc2, w2s, w2n, wfc)


def _edge_body(table_hbm, src_hbm, dst_hbm, zeros_hbm, out_hbm,
               srcv, dstv, buf0, buf1, acc_sh, gsem0, gsem1):
    cid = lax.axis_index("c")
    sid = lax.axis_index("s")
    rbase = sid * _RPT
    crow = (cid * _NS + sid) * _NCHUNKS
    pltpu.sync_copy(src_hbm.at[pl.ds(crow, _NCHUNKS)], srcv)
    pltpu.sync_copy(dst_hbm.at[pl.ds(crow, _NCHUNKS)], dstv)
    pltpu.sync_copy(zeros_hbm.at[pl.ds(rbase, _RPT)],
                    acc_sh.at[pl.ds(rbase, _RPT)])
    plsc.subcore_barrier()

    pltpu.async_copy(table_hbm.at[srcv.at[0]], buf0, gsem0)

    def pair(k, carry):
        i0 = 2 * k
        pltpu.make_async_copy(table_hbm.at[srcv.at[i0]], buf0, gsem0).wait()
        pltpu.async_copy(table_hbm.at[srcv.at[i0 + 1]], buf1, gsem1)
        pltpu.sync_copy(buf0, acc_sh.at[dstv.at[i0]], add=True)
        pltpu.make_async_copy(table_hbm.at[srcv.at[i0 + 1]], buf1, gsem1).wait()
        pltpu.async_copy(table_hbm.at[srcv.at[i0 + 2]], buf0, gsem0)
        pltpu.sync_copy(buf1, acc_sh.at[dstv.at[i0 + 1]], add=True)
        return carry

    lax.fori_loop(0, (_NCHUNKS - 1) // 2, pair, 0)
    last = _NCHUNKS - 1
    pltpu.make_async_copy(table_hbm.at[srcv.at[last]], buf0, gsem0).wait()
    pltpu.sync_copy(buf0, acc_sh.at[dstv.at[last]], add=True)

    plsc.subcore_barrier()
    pltpu.sync_copy(acc_sh.at[pl.ds(rbase, _RPT)],
                    out_hbm.at[pl.ds(cid * _NPAD + rbase, _RPT)])


@functools.cache
def _edge_pass_call():
    return pl.kernel(
        _edge_body,
        out_type=jax.ShapeDtypeStruct((_NC * _NPAD, _L), jnp.float32),
        mesh=plsc.VectorSubcoreMesh(core_axis_name="c", subcore_axis_name="s",
                                    num_cores=_NC, num_subcores=_NS),
        scratch_types=[
            pltpu.VMEM((_NCHUNKS, _CHUNK), jnp.int32),
            pltpu.VMEM((_NCHUNKS, _CHUNK), jnp.int32),
            pltpu.VMEM((_CHUNK, _L), jnp.float32),
            pltpu.VMEM((_CHUNK, _L), jnp.float32),
            pltpu.VMEM_SHARED((_NPAD, _L), jnp.float32),
            pltpu.SemaphoreType.DMA,
            pltpu.SemaphoreType.DMA,
        ],
        compiler_params=pltpu.CompilerParams(use_tc_tiling_on_sc=False),
    )


def _edge_pass(table, src2d, dst2d, ztbl):
    return _edge_pass_call()(table, src2d, dst2d, ztbl)


def kernel(features, edge_index, w1_self, w1_neigh, w2_self, w2_neigh, w_fc1):
    src = edge_index[0].reshape(_E // _CHUNK, _CHUNK)
    dst = edge_index[1].reshape(_E // _CHUNK, _CHUNK)
    feat_pad = jnp.pad(features, ((0, _NPAD - _N), (0, 0)))
    wn = jnp.pad(w1_neigh, ((0, 0), (0, _L - _H)))
    ws = jnp.pad(w1_self, ((0, 0), (0, _L - _H)))
    w2s = jnp.pad(w2_self, ((0, _L - _H), (0, _L - _H)))
    w2n = jnp.pad(w2_neigh, ((0, _L - _H), (0, _L - _H)))
    wfc = jnp.pad(w_fc1, ((0, _L - _H), (0, 127)))
    ztbl = jnp.zeros((_NPAD, _L), jnp.float32)

    pn, ps = _project(feat_pad, wn, ws)
    acc1 = _edge_pass(pn, src, dst, ztbl).reshape(_NC, _NPAD, _L)
    x1 = _x1(ps, acc1)
    acc2 = _edge_pass(x1, src, dst, ztbl).reshape(_NC, _NPAD, _L)
    return _final(x1, acc1, acc2, w2s, w2n, wfc)


# speedup vs baseline: 22.1237x; 1.6480x over previous
import functools

import jax
import jax.numpy as jnp
from jax import lax
from jax.experimental import pallas as pl
from jax.experimental.pallas import tpu as pltpu
from jax.experimental.pallas import tpu_sc as plsc

_N = 10000
_E = 320000
_D = 128
_H = 10
_L = 16
_NC = 2
_NS = 16
_NW = _NC * _NS
_NPAD = 10240
_RPT = _NPAD // _NS
_EPW = _E // _NW
_CHUNK = 80
_NCHUNKS = _EPW // _CHUNK


def _project_body(feat_ref, wn_ref, ws_ref, pn_ref, ps_ref):
    f = feat_ref[...]
    pn = jnp.dot(f, wn_ref[...], preferred_element_type=jnp.float32)
    ps = jnp.dot(f, ws_ref[...], preferred_element_type=jnp.float32)
    lane = lax.broadcasted_iota(jnp.int32, pn.shape, 1)
    pn_ref[...] = jnp.where(lane == _H, 1.0, pn)
    ps_ref[...] = ps


def _project(feat_pad, wn, ws):
    bm = 1024
    return pl.pallas_call(
        _project_body,
        grid=(_NPAD // bm,),
        in_specs=[
            pl.BlockSpec((bm, _D), lambda i: (i, 0)),
            pl.BlockSpec((_D, _L), lambda i: (0, 0)),
            pl.BlockSpec((_D, _L), lambda i: (0, 0)),
        ],
        out_specs=[
            pl.BlockSpec((bm, _L), lambda i: (i, 0)),
            pl.BlockSpec((bm, _L), lambda i: (i, 0)),
        ],
        out_shape=[
            jax.ShapeDtypeStruct((_NPAD, _L), jnp.float32),
            jax.ShapeDtypeStruct((_NPAD, _L), jnp.float32),
        ],
    )(feat_pad, wn, ws)


def _readout_body(parts_ref, w2s_ref, w2n_ref, wfc_ref, out_ref):
    a_vec = parts_ref[0:1, 0:_L]
    b_vec = parts_ref[0:1, _L:2 * _L] + parts_ref[1:2, _L:2 * _L]
    g = (jnp.dot(a_vec, w2s_ref[...], preferred_element_type=jnp.float32)
         + jnp.dot(b_vec, w2n_ref[...], preferred_element_type=jnp.float32))
    g = g * (1.0 / _N)
    o = jnp.dot(g, wfc_ref[...], preferred_element_type=jnp.float32)
    out_ref[...] = jax.nn.sigmoid(o[:, :1])


def _readout(parts, w2s, w2n, wfc):
    return pl.pallas_call(
        _readout_body,
        out_shape=jax.ShapeDtypeStruct((1, 1), jnp.float32),
    )(parts, w2s, w2n, wfc)


def _ring_loop(table_hbm, srcv, dstv, acc_sh, bufs, gsems, ssems):
    def step(i, b, first_round):
        pltpu.make_async_copy(table_hbm.at[srcv.at[i]], bufs[b], gsems[b]).wait()
        pltpu.async_copy(bufs[b], acc_sh.at[dstv.at[i]], ssems[b], add=True)
        nxt = i + 2
        bn = (b + 2) % 4
        if not first_round:
            pltpu.make_async_copy(bufs[bn], acc_sh.at[dstv.at[nxt - 4]],
                                  ssems[bn]).wait()
        pltpu.async_copy(table_hbm.at[srcv.at[nxt]], bufs[bn], gsems[bn])

    pltpu.async_copy(table_hbm.at[srcv.at[0]], bufs[0], gsems[0])
    pltpu.async_copy(table_hbm.at[srcv.at[1]], bufs[1], gsems[1])
    step(0, 0, True)
    step(1, 1, True)

    def group(k, carry):
        i0 = 4 * k + 2
        step(i0, 2, False)
        step(i0 + 1, 3, False)
        step(i0 + 2, 0, False)
        step(i0 + 3, 1, False)
        return carry

    lax.fori_loop(0, (_NCHUNKS - 5) // 4, group, 0)
    step(_NCHUNKS - 3, 2, False)
    pltpu.make_async_copy(table_hbm.at[srcv.at[_NCHUNKS - 2]], bufs[3],
                          gsems[3]).wait()
    pltpu.async_copy(bufs[3], acc_sh.at[dstv.at[_NCHUNKS - 2]], ssems[3],
                     add=True)
    pltpu.make_async_copy(table_hbm.at[srcv.at[_NCHUNKS - 1]], bufs[0],
                          gsems[0]).wait()
    pltpu.async_copy(bufs[0], acc_sh.at[dstv.at[_NCHUNKS - 1]], ssems[0],
                     add=True)
    for b in (1, 2, 3, 0):
        pltpu.make_async_copy(bufs[b], acc_sh.at[dstv.at[0]], ssems[b]).wait()


def _pass1_body(table_hbm, e_hbm, zeros_hbm, out_hbm,
                srcv, dstv, buf0, buf1, buf2, buf3, acc_sh,
                gsem0, gsem1, gsem2, gsem3, ssem0, ssem1, ssem2, ssem3):
    cid = lax.axis_index("c")
    sid = lax.axis_index("s")
    rbase = sid * _RPT
    crow = (cid * _NS + sid) * _NCHUNKS
    pltpu.sync_copy(e_hbm.at[0, pl.ds(crow, _NCHUNKS)], srcv)
    pltpu.sync_copy(e_hbm.at[1, pl.ds(crow, _NCHUNKS)], dstv)
    pltpu.sync_copy(zeros_hbm.at[pl.ds(rbase, _RPT)],
                    acc_sh.at[pl.ds(rbase, _RPT)])
    plsc.subcore_barrier()
    _ring_loop(table_hbm, srcv, dstv, acc_sh,
               (buf0, buf1, buf2, buf3),
               (gsem0, gsem1, gsem2, gsem3),
               (ssem0, ssem1, ssem2, ssem3))
    plsc.subcore_barrier()
    pltpu.sync_copy(acc_sh.at[pl.ds(rbase, _RPT)],
                    out_hbm.at[pl.ds(cid * _NPAD + rbase, _RPT)])


def _mega_body(ps_hbm, acc1_hbm, e_hbm, zeros_hbm, parts_hbm, x1_hbm,
               srcv, dstv, buf0, buf1, buf2, buf3,
               psv, av0, av1, iv, xv, pv, pall, acc_sh, parts_sh,
               gsem0, gsem1, gsem2, gsem3, ssem0, ssem1, ssem2, ssem3):
    cid = lax.axis_index("c")
    sid = lax.axis_index("s")
    rbase = sid * _RPT
    crow = (cid * _NS + sid) * _NCHUNKS
    pltpu.sync_copy(e_hbm.at[0, pl.ds(crow, _NCHUNKS)], srcv)
    pltpu.sync_copy(e_hbm.at[1, pl.ds(crow, _NCHUNKS)], dstv)
    pltpu.sync_copy(ps_hbm.at[pl.ds(rbase, _RPT)], psv)
    pltpu.sync_copy(acc1_hbm.at[pl.ds(rbase, _RPT)], av0)
    pltpu.sync_copy(acc1_hbm.at[pl.ds(_NPAD + rbase, _RPT)], av1)
    pltpu.sync_copy(zeros_hbm.at[pl.ds(rbase, _RPT)],
                    acc_sh.at[pl.ds(rbase, _RPT)])

    iota16 = lax.broadcasted_iota(jnp.int32, (_L,), 0)

    def _row(ref, r):
        return plsc.load_gather(ref, [jnp.full((_L,), r, jnp.int32), iota16])

    def _setrow(ref, r, x):
        plsc.store_scatter(ref, [jnp.full((_L,), r, jnp.int32), iota16], x)

    off = cid * _NPAD

    def offrow(i, carry):
        ir = jnp.full((_L,), i, jnp.int32)
        for j in range(_CHUNK // _L):
            ic = iota16 + (j * _L)
            plsc.store_scatter(srcv, [ir, ic],
                               plsc.load_gather(srcv, [ir, ic]) + off)
        return carry

    lax.fori_loop(0, _NCHUNKS, offrow, 0)

    mask10 = iota16 < _H

    def xrow(r, apart):
        arow = _row(av0, r) + _row(av1, r)
        degv = jnp.broadcast_to(arow[_H], (_L,))
        invd = 1.0 / jnp.maximum(degv, 1.0)
        x1r = jnp.maximum(_row(psv, r) + arow * invd, 0.0)
        x1r = jnp.where(mask10, x1r, 0.0)
        _setrow(xv, r, x1r)
        _setrow(iv, r, invd)
        return apart + x1r

    apart = lax.fori_loop(0, _RPT, xrow, jnp.zeros((_L,), jnp.float32))
    pltpu.sync_copy(xv, x1_hbm.at[pl.ds(off + rbase, _RPT)])
    plsc.subcore_barrier()

    _ring_loop(x1_hbm, srcv, dstv, acc_sh,
               (buf0, buf1, buf2, buf3),
               (gsem0, gsem1, gsem2, gsem3),
               (ssem0, ssem1, ssem2, ssem3))
    plsc.subcore_barrier()

    pltpu.sync_copy(acc_sh.at[pl.ds(rbase, _RPT)], av0)

    def brow(r, bpart):
        return bpart + _row(av0, r) * _row(iv, r)

    bpart = lax.fori_loop(0, _RPT, brow, jnp.zeros((_L,), jnp.float32))
    pv[pl.ds(0, _L)] = apart
    pv[pl.ds(_L, _L)] = bpart
    pltpu.sync_copy(pv, parts_sh.at[sid])
    plsc.subcore_barrier()

    @pl.when(sid == 0)
    def _():
        pltpu.sync_copy(parts_sh, pall)

        def red(t, ab):
            tr = jnp.full((_L,), t, jnp.int32)
            pa = plsc.load_gather(pall, [tr, iota16])
            pb = plsc.load_gather(pall, [tr, iota16 + _L])
            return (ab[0] + pa, ab[1] + pb)

        asum, bsum = lax.fori_loop(
            0, _NS, red,
            (jnp.zeros((_L,), jnp.float32), jnp.zeros((_L,), jnp.float32)))
        pv[pl.ds(0, _L)] = asum
        pv[pl.ds(_L, _L)] = bsum
        pltpu.sync_copy(pv, parts_hbm.at[cid])


_SC_PARAMS = pltpu.CompilerParams(use_tc_tiling_on_sc=False,
                                  needs_layout_passes=False)


@functools.cache
def _pass1_call():
    return pl.kernel(
        _pass1_body,
        out_type=jax.ShapeDtypeStruct((_NC * _NPAD, _L), jnp.float32),
        mesh=plsc.VectorSubcoreMesh(core_axis_name="c", subcore_axis_name="s",
                                    num_cores=_NC, num_subcores=_NS),
        scratch_types=[
            pltpu.VMEM((_NCHUNKS, _CHUNK), jnp.int32),
            pltpu.VMEM((_NCHUNKS, _CHUNK), jnp.int32),
            pltpu.VMEM((_CHUNK, _L), jnp.float32),
            pltpu.VMEM((_CHUNK, _L), jnp.float32),
            pltpu.VMEM((_CHUNK, _L), jnp.float32),
            pltpu.VMEM((_CHUNK, _L), jnp.float32),
            pltpu.VMEM_SHARED((_NPAD, _L), jnp.float32),
        ] + [pltpu.SemaphoreType.DMA] * 8,
        compiler_params=_SC_PARAMS,
    )


@functools.cache
def _mega_call():
    return pl.kernel(
        _mega_body,
        out_type=[
            jax.ShapeDtypeStruct((_NC, 2 * _L), jnp.float32),
            jax.ShapeDtypeStruct((_NC * _NPAD, _L), jnp.float32),
        ],
        mesh=plsc.VectorSubcoreMesh(core_axis_name="c", subcore_axis_name="s",
                                    num_cores=_NC, num_subcores=_NS),
        scratch_types=[
            pltpu.VMEM((_NCHUNKS, _CHUNK), jnp.int32),
            pltpu.VMEM((_NCHUNKS, _CHUNK), jnp.int32),
            pltpu.VMEM((_CHUNK, _L), jnp.float32),
            pltpu.VMEM((_CHUNK, _L), jnp.float32),
            pltpu.VMEM((_CHUNK, _L), jnp.float32),
            pltpu.VMEM((_CHUNK, _L), jnp.float32),
            pltpu.VMEM((_RPT, _L), jnp.float32),
            pltpu.VMEM((_RPT, _L), jnp.float32),
            pltpu.VMEM((_RPT, _L), jnp.float32),
            pltpu.VMEM((_RPT, _L), jnp.float32),
            pltpu.VMEM((_RPT, _L), jnp.float32),
            pltpu.VMEM((2 * _L,), jnp.float32),
            pltpu.VMEM((_NS, 2 * _L), jnp.float32),
            pltpu.VMEM_SHARED((_NPAD, _L), jnp.float32),
            pltpu.VMEM_SHARED((_NS, 2 * _L), jnp.float32),
        ] + [pltpu.SemaphoreType.DMA] * 8,
        compiler_params=_SC_PARAMS,
    )


def _pass1(table, e3, ztbl):
    return _pass1_call()(table, e3, ztbl)


def _mega(ps, acc1, e3, ztbl):
    return _mega_call()(ps, acc1, e3, ztbl)


def kernel(features, edge_index, w1_self, w1_neigh, w2_self, w2_neigh, w_fc1):
    e3 = edge_index.reshape(2, _E // _CHUNK, _CHUNK)
    feat_pad = jnp.pad(features, ((0, _NPAD - _N), (0, 0)))
    wn = jnp.pad(w1_neigh, ((0, 0), (0, _L - _H)))
    ws = jnp.pad(w1_self, ((0, 0), (0, _L - _H)))
    w2s = jnp.pad(w2_self, ((0, _L - _H), (0, _L - _H)))
    w2n = jnp.pad(w2_neigh, ((0, _L - _H), (0, _L - _H)))
    wfc = jnp.pad(w_fc1, ((0, _L - _H), (0, _D - 1)))
    ztbl = jnp.zeros((_NPAD, _L), jnp.float32)

    pn, ps = _project(feat_pad, wn, ws)
    acc1 = _pass1(pn, e3, ztbl)
    parts, _ = _mega(ps, acc1, e3, ztbl)
    return _readout(parts, w2s, w2n, wfc)


# speedup vs baseline: 23.0375x; 1.0413x over previous
import functools

import jax
import jax.numpy as jnp
from jax import lax
from jax.experimental import pallas as pl
from jax.experimental.pallas import tpu as pltpu
from jax.experimental.pallas import tpu_sc as plsc

_N = 10000
_E = 320000
_D = 128
_H = 10
_L = 16
_NC = 2
_NS = 16
_NW = _NC * _NS
_NPAD = 10240
_RPT = _NPAD // _NS
_EPW = _E // _NW
_CHUNK = 80
_NCHUNKS = _EPW // _CHUNK


def _project_n_body(feat_ref, w_ref, out_ref):
    p = jnp.dot(feat_ref[...], w_ref[...], preferred_element_type=jnp.float32)
    lane = lax.broadcasted_iota(jnp.int32, p.shape, 1)
    out_ref[...] = jnp.where(lane == _H, 1.0, p)


def _project_s_body(feat_ref, w_ref, out_ref):
    out_ref[...] = jnp.dot(feat_ref[...], w_ref[...],
                           preferred_element_type=jnp.float32)


def _project(feat, w, body):
    bm = 2000
    return pl.pallas_call(
        body,
        grid=(_N // bm,),
        in_specs=[
            pl.BlockSpec((bm, _D), lambda i: (i, 0)),
            pl.BlockSpec((_D, _L), lambda i: (0, 0)),
        ],
        out_specs=pl.BlockSpec((bm, _L), lambda i: (i, 0)),
        out_shape=jax.ShapeDtypeStruct((_N, _L), jnp.float32),
    )(feat, w)


def _readout_body(parts_ref, w2s_ref, w2n_ref, wfc_ref, out_ref):
    a_vec = parts_ref[0:1, 0:_L]
    b_vec = parts_ref[0:1, _L:2 * _L] + parts_ref[1:2, _L:2 * _L]
    g = (jnp.dot(a_vec, w2s_ref[...], preferred_element_type=jnp.float32)
         + jnp.dot(b_vec, w2n_ref[...], preferred_element_type=jnp.float32))
    g = g * (1.0 / _N)
    o = jnp.dot(g, wfc_ref[...], preferred_element_type=jnp.float32)
    out_ref[...] = jax.nn.sigmoid(o[:, :1])


def _readout(parts, w2s, w2n, wfc):
    return pl.pallas_call(
        _readout_body,
        out_shape=jax.ShapeDtypeStruct((1, 1), jnp.float32),
    )(parts, w2s, w2n, wfc)


def _ring_loop(table_hbm, srcv, dstv, acc_sh, bufs, gsems, ssems):
    def step(i, b, first_round):
        pltpu.make_async_copy(table_hbm.at[srcv.at[i]], bufs[b], gsems[b]).wait()
        pltpu.async_copy(bufs[b], acc_sh.at[dstv.at[i]], ssems[b], add=True)
        nxt = i + 2
        bn = (b + 2) % 4
        if not first_round:
            pltpu.make_async_copy(bufs[bn], acc_sh.at[dstv.at[nxt - 4]],
                                  ssems[bn]).wait()
        pltpu.async_copy(table_hbm.at[srcv.at[nxt]], bufs[bn], gsems[bn])

    pltpu.async_copy(table_hbm.at[srcv.at[0]], bufs[0], gsems[0])
    pltpu.async_copy(table_hbm.at[srcv.at[1]], bufs[1], gsems[1])
    step(0, 0, True)
    step(1, 1, True)

    def group(k, carry):
        i0 = 4 * k + 2
        step(i0, 2, False)
        step(i0 + 1, 3, False)
        step(i0 + 2, 0, False)
        step(i0 + 3, 1, False)
        return carry

    lax.fori_loop(0, (_NCHUNKS - 5) // 4, group, 0)
    step(_NCHUNKS - 3, 2, False)
    pltpu.make_async_copy(table_hbm.at[srcv.at[_NCHUNKS - 2]], bufs[3],
                          gsems[3]).wait()
    pltpu.async_copy(bufs[3], acc_sh.at[dstv.at[_NCHUNKS - 2]], ssems[3],
                     add=True)
    pltpu.make_async_copy(table_hbm.at[srcv.at[_NCHUNKS - 1]], bufs[0],
                          gsems[0]).wait()
    pltpu.async_copy(bufs[0], acc_sh.at[dstv.at[_NCHUNKS - 1]], ssems[0],
                     add=True)
    for b in (1, 2, 3, 0):
        pltpu.make_async_copy(bufs[b], acc_sh.at[dstv.at[0]], ssems[b]).wait()


def _pass1_body(table_hbm, e_hbm, zeros_hbm, out_hbm,
                srcv, dstv, buf0, buf1, buf2, buf3, acc_sh,
                gsem0, gsem1, gsem2, gsem3, ssem0, ssem1, ssem2, ssem3):
    cid = lax.axis_index("c")
    sid = lax.axis_index("s")
    rbase = sid * _RPT
    crow = (cid * _NS + sid) * _NCHUNKS
    pltpu.sync_copy(e_hbm.at[0, pl.ds(crow, _NCHUNKS)], srcv)
    pltpu.sync_copy(e_hbm.at[1, pl.ds(crow, _NCHUNKS)], dstv)
    pltpu.sync_copy(zeros_hbm.at[pl.ds(rbase, _RPT)],
                    acc_sh.at[pl.ds(rbase, _RPT)])
    plsc.subcore_barrier()
    _ring_loop(table_hbm, srcv, dstv, acc_sh,
               (buf0, buf1, buf2, buf3),
               (gsem0, gsem1, gsem2, gsem3),
               (ssem0, ssem1, ssem2, ssem3))
    plsc.subcore_barrier()
    pltpu.sync_copy(acc_sh.at[pl.ds(rbase, _RPT)],
                    out_hbm.at[pl.ds(cid * _NPAD + rbase, _RPT)])


def _mega_body(ps_hbm, acc1_hbm, e_hbm, zeros_hbm, parts_hbm, x1_hbm,
               srcv, dstv, buf0, buf1, buf2, buf3,
               psv, av0, av1, iv, xv, pv, pall, acc_sh, parts_sh,
               gsem0, gsem1, gsem2, gsem3, ssem0, ssem1, ssem2, ssem3):
    cid = lax.axis_index("c")
    sid = lax.axis_index("s")
    rbase = sid * _RPT
    crow = (cid * _NS + sid) * _NCHUNKS
    pltpu.sync_copy(e_hbm.at[0, pl.ds(crow, _NCHUNKS)], srcv)
    pltpu.sync_copy(e_hbm.at[1, pl.ds(crow, _NCHUNKS)], dstv)
    pltpu.sync_copy(ps_hbm.at[pl.ds(rbase, _RPT)], psv)
    pltpu.sync_copy(acc1_hbm.at[pl.ds(rbase, _RPT)], av0)
    pltpu.sync_copy(acc1_hbm.at[pl.ds(_NPAD + rbase, _RPT)], av1)
    pltpu.sync_copy(zeros_hbm.at[pl.ds(rbase, _RPT)],
                    acc_sh.at[pl.ds(rbase, _RPT)])

    iota16 = lax.broadcasted_iota(jnp.int32, (_L,), 0)

    def _row(ref, r):
        return plsc.load_gather(ref, [jnp.full((_L,), r, jnp.int32), iota16])

    def _setrow(ref, r, x):
        plsc.store_scatter(ref, [jnp.full((_L,), r, jnp.int32), iota16], x)

    off = cid * _NPAD

    def offrow(i, carry):
        ir = jnp.full((_L,), i, jnp.int32)
        for j in range(_CHUNK // _L):
            ic = iota16 + (j * _L)
            plsc.store_scatter(srcv, [ir, ic],
                               plsc.load_gather(srcv, [ir, ic]) + off)
        return carry

    lax.fori_loop(0, _NCHUNKS, offrow, 0)

    mask10 = iota16 < _H

    def xrow(r, apart):
        arow = _row(av0, r) + _row(av1, r)
        degv = jnp.broadcast_to(arow[_H], (_L,))
        invd = 1.0 / jnp.maximum(degv, 1.0)
        x1r = jnp.maximum(_row(psv, r) + arow * invd, 0.0)
        x1r = jnp.where(mask10, x1r, 0.0)
        _setrow(xv, r, x1r)
        _setrow(iv, r, invd)
        return apart + x1r

    apart = lax.fori_loop(0, _RPT, xrow, jnp.zeros((_L,), jnp.float32))
    pltpu.sync_copy(xv, x1_hbm.at[pl.ds(off + rbase, _RPT)])
    plsc.subcore_barrier()

    _ring_loop(x1_hbm, srcv, dstv, acc_sh,
               (buf0, buf1, buf2, buf3),
               (gsem0, gsem1, gsem2, gsem3),
               (ssem0, ssem1, ssem2, ssem3))
    plsc.subcore_barrier()

    pltpu.sync_copy(acc_sh.at[pl.ds(rbase, _RPT)], av0)

    def brow(r, bpart):
        return bpart + _row(av0, r) * _row(iv, r)

    bpart = lax.fori_loop(0, _RPT, brow, jnp.zeros((_L,), jnp.float32))
    pv[pl.ds(0, _L)] = apart
    pv[pl.ds(_L, _L)] = bpart
    pltpu.sync_copy(pv, parts_sh.at[sid])
    plsc.subcore_barrier()

    @pl.when(sid == 0)
    def _():
        pltpu.sync_copy(parts_sh, pall)

        def red(t, ab):
            tr = jnp.full((_L,), t, jnp.int32)
            pa = plsc.load_gather(pall, [tr, iota16])
            pb = plsc.load_gather(pall, [tr, iota16 + _L])
            return (ab[0] + pa, ab[1] + pb)

        asum, bsum = lax.fori_loop(
            0, _NS, red,
            (jnp.zeros((_L,), jnp.float32), jnp.zeros((_L,), jnp.float32)))
        pv[pl.ds(0, _L)] = asum
        pv[pl.ds(_L, _L)] = bsum
        pltpu.sync_copy(pv, parts_hbm.at[cid])


_SC_PARAMS = pltpu.CompilerParams(use_tc_tiling_on_sc=False,
                                  needs_layout_passes=False)


@functools.cache
def _pass1_call():
    return pl.kernel(
        _pass1_body,
        out_type=jax.ShapeDtypeStruct((_NC * _NPAD, _L), jnp.float32),
        mesh=plsc.VectorSubcoreMesh(core_axis_name="c", subcore_axis_name="s",
                                    num_cores=_NC, num_subcores=_NS),
        scratch_types=[
            pltpu.VMEM((_NCHUNKS, _CHUNK), jnp.int32),
            pltpu.VMEM((_NCHUNKS, _CHUNK), jnp.int32),
            pltpu.VMEM((_CHUNK, _L), jnp.float32),
            pltpu.VMEM((_CHUNK, _L), jnp.float32),
            pltpu.VMEM((_CHUNK, _L), jnp.float32),
            pltpu.VMEM((_CHUNK, _L), jnp.float32),
            pltpu.VMEM_SHARED((_NPAD, _L), jnp.float32),
        ] + [pltpu.SemaphoreType.DMA] * 8,
        compiler_params=_SC_PARAMS,
    )


@functools.cache
def _mega_call():
    return pl.kernel(
        _mega_body,
        out_type=[
            jax.ShapeDtypeStruct((_NC, 2 * _L), jnp.float32),
            jax.ShapeDtypeStruct((_NC * _NPAD, _L), jnp.float32),
        ],
        mesh=plsc.VectorSubcoreMesh(core_axis_name="c", subcore_axis_name="s",
                                    num_cores=_NC, num_subcores=_NS),
        scratch_types=[
            pltpu.VMEM((_NCHUNKS, _CHUNK), jnp.int32),
            pltpu.VMEM((_NCHUNKS, _CHUNK), jnp.int32),
            pltpu.VMEM((_CHUNK, _L), jnp.float32),
            pltpu.VMEM((_CHUNK, _L), jnp.float32),
            pltpu.VMEM((_CHUNK, _L), jnp.float32),
            pltpu.VMEM((_CHUNK, _L), jnp.float32),
            pltpu.VMEM((_RPT, _L), jnp.float32),
            pltpu.VMEM((_RPT, _L), jnp.float32),
            pltpu.VMEM((_RPT, _L), jnp.float32),
            pltpu.VMEM((_RPT, _L), jnp.float32),
            pltpu.VMEM((_RPT, _L), jnp.float32),
            pltpu.VMEM((2 * _L,), jnp.float32),
            pltpu.VMEM((_NS, 2 * _L), jnp.float32),
            pltpu.VMEM_SHARED((_NPAD, _L), jnp.float32),
            pltpu.VMEM_SHARED((_NS, 2 * _L), jnp.float32),
        ] + [pltpu.SemaphoreType.DMA] * 8,
        compiler_params=_SC_PARAMS,
    )


def _pass1(table, e3, ztbl):
    return _pass1_call()(table, e3, ztbl)


def _mega(ps, acc1, e3, ztbl):
    return _mega_call()(ps, acc1, e3, ztbl)


def kernel(features, edge_index, w1_self, w1_neigh, w2_self, w2_neigh, w_fc1):
    e3 = edge_index.reshape(2, _E // _CHUNK, _CHUNK)
    wn = jnp.pad(w1_neigh, ((0, 0), (0, _L - _H)))
    ws = jnp.pad(w1_self, ((0, 0), (0, _L - _H)))
    w2s = jnp.pad(w2_self, ((0, _L - _H), (0, _L - _H)))
    w2n = jnp.pad(w2_neigh, ((0, _L - _H), (0, _L - _H)))
    wfc = jnp.pad(w_fc1, ((0, _L - _H), (0, _D - 1)))
    ztbl = jnp.zeros((_NPAD, _L), jnp.float32)

    pn = _project(features, wn, _project_n_body)
    acc1 = _pass1(pn, e3, ztbl)
    ps = jnp.pad(_project(features, ws, _project_s_body),
                 ((0, _NPAD - _N), (0, 0)))
    parts, _ = _mega(ps, acc1, e3, ztbl)
    return _readout(parts, w2s, w2n, wfc)
